# baseline (speedup 1.0000x reference)
import jax
import jax.numpy as jnp
from jax.experimental import pallas as pl
from jax.experimental.pallas import tpu as pltpu

_B, _S, _D = 4, 8192, 1024
_CH = 2048
_NCH = _S // _CH


def _dma_kernel(w_hbm, o_hbm, sem):
    copies = []
    for c in range(_NCH):
        for b in range(_B):
            copies.append(pltpu.async_copy(
                w_hbm.at[pl.ds(c * _CH, _CH)],
                o_hbm.at[b, pl.ds(c * _CH, _CH)],
                sem))
    for h in copies:
        h.wait()


def kernel(x, weight):
    return pl.pallas_call(
        _dma_kernel,
        in_specs=[pl.BlockSpec(memory_space=pl.ANY)],
        out_specs=pl.BlockSpec(memory_space=pl.ANY),
        out_shape=jax.ShapeDtypeStruct((_B, _S, _D), jnp.float32),
        scratch_shapes=[pltpu.SemaphoreType.DMA],
    )(weight)


# speedup vs baseline: 72.9967x; 72.9967x over previous
import jax
import jax.numpy as jnp
from jax.experimental import pallas as pl
from jax.experimental.pallas import tpu as pltpu

_B, _S, _D = 4, 8192, 1024
_CH = 512
_NCHUNK = _S // _CH
_NBUF = 3


def _ring_body(w_hbm, o_hbm, b0, b1, b2, r0, r1, r2, w0, w1, w2):
    bufs = (b0, b1, b2)
    rsems = (r0, r1, r2)
    wsems = (w0, w1, w2)

    def start_read(i):
        return pltpu.async_copy(
            w_hbm.at[pl.ds(i * _CH, _CH)], bufs[i % _NBUF], rsems[i % _NBUF])

    reads = {0: start_read(0)}
    writes = {}
    for i in range(_NCHUNK):
        reads.pop(i).wait()
        writes[i] = [
            pltpu.async_copy(
                bufs[i % _NBUF], o_hbm.at[b, pl.ds(i * _CH, _CH)],
                wsems[i % _NBUF])
            for b in range(_B)
        ]
        if i - 2 >= 0:
            for h in writes.pop(i - 2):
                h.wait()
        if i + 1 < _NCHUNK:
            reads[i + 1] = start_read(i + 1)
    for i in (_NCHUNK - 2, _NCHUNK - 1):
        for h in writes.pop(i, []):
            h.wait()


def kernel(x, weight):
    return pl.pallas_call(
        _ring_body,
        in_specs=[pl.BlockSpec(memory_space=pl.ANY)],
        out_specs=pl.BlockSpec(memory_space=pl.ANY),
        out_shape=jax.ShapeDtypeStruct((_B, _S, _D), jnp.float32),
        scratch_shapes=[
            pltpu.VMEM((_CH, _D), jnp.float32),
            pltpu.VMEM((_CH, _D), jnp.float32),
            pltpu.VMEM((_CH, _D), jnp.float32),
            pltpu.SemaphoreType.DMA,
            pltpu.SemaphoreType.DMA,
            pltpu.SemaphoreType.DMA,
            pltpu.SemaphoreType.DMA,
            pltpu.SemaphoreType.DMA,
            pltpu.SemaphoreType.DMA,
        ],
    )(weight)
